# Initial kernel scaffold; baseline (speedup 1.0000x reference)
import jax
import jax.numpy as jnp
from jax.experimental import pallas as pl


def kernel(T, segment_ids, X_pseudo, weights):
    raise NotImplementedError("write your pallas kernel here")


# speedup vs baseline: 5.2629x; 5.2629x over previous
import functools

import jax
import jax.numpy as jnp
from jax import lax
from jax.experimental import pallas as pl
from jax.experimental.pallas import tpu as pltpu
from jax.experimental.pallas import tpu_sc as plsc

N_ROWS = 320000
D_FEAT = 128
N_SEG = 10000
N_SEG_PAD = 10240

ROW_BLOCK = 8000

NUM_TILES = 16
WINDOW = 128
W_PER_TILE = 160
N_WINDOWS = NUM_TILES * W_PER_TILE
N_ROWS_PAD = N_WINDOWS * WINDOW
DUMMY_SEG = N_SEG + 8


def _matvec_body(t_ref, x_ref, w_ref, s_ref):
    v_row = jnp.dot(w_ref[...], x_ref[...], preferred_element_type=jnp.float32)
    s_ref[...] = jnp.dot(t_ref[...], v_row.T, preferred_element_type=jnp.float32)


def _rowdot(T, X_pseudo, weights):
    return pl.pallas_call(
        _matvec_body,
        grid=(N_ROWS // ROW_BLOCK,),
        in_specs=[
            pl.BlockSpec((ROW_BLOCK, D_FEAT), lambda i: (i, 0)),
            pl.BlockSpec((512, D_FEAT), lambda i: (0, 0)),
            pl.BlockSpec((1, 512), lambda i: (0, 0)),
        ],
        out_specs=pl.BlockSpec((ROW_BLOCK, 1), lambda i: (i, 0)),
        out_shape=jax.ShapeDtypeStruct((N_ROWS, 1), jnp.float32),
    )(T, X_pseudo, weights)


def _segsum_tec(ids_hbm, s_hbm, out_hbm, idx_v, upd_v, zero_v, stage_v, acc_shared):
    tid = lax.axis_index("s")

    for i in range(640 // 16):
        zero_v[pl.ds(i * 16, 16)] = jnp.zeros((16,), jnp.float32)
    pltpu.sync_copy(zero_v, acc_shared.at[pl.ds(tid * 640, 640)])
    plsc.subcore_barrier()

    pltpu.sync_copy(ids_hbm.at[pl.ds(tid * W_PER_TILE, W_PER_TILE)], idx_v)
    pltpu.sync_copy(s_hbm.at[pl.ds(tid * W_PER_TILE, W_PER_TILE)], upd_v)

    def body(w, carry):
        pltpu.sync_copy(upd_v.at[w], acc_shared.at[idx_v.at[w]], add=True)
        return carry

    lax.fori_loop(0, W_PER_TILE, body, 0)
    plsc.subcore_barrier()

    @pl.when(tid == 0)
    def _():
        pltpu.sync_copy(acc_shared.at[pl.ds(0, N_SEG)], stage_v)
        pltpu.sync_copy(stage_v, out_hbm)


def _segment_sum_sc(ids2d, s2d):
    mesh = plsc.VectorSubcoreMesh(
        core_axis_name="c", subcore_axis_name="s", num_cores=1
    )
    f = pl.kernel(
        _segsum_tec,
        out_type=jax.ShapeDtypeStruct((N_SEG,), jnp.float32),
        mesh=mesh,
        scratch_types=[
            pltpu.VMEM((W_PER_TILE, WINDOW), jnp.int32),
            pltpu.VMEM((W_PER_TILE, WINDOW), jnp.float32),
            pltpu.VMEM((640,), jnp.float32),
            pltpu.VMEM((N_SEG,), jnp.float32),
            pltpu.VMEM_SHARED((N_SEG_PAD,), jnp.float32),
        ],
    )
    return f(ids2d, s2d)


def kernel(T, segment_ids, X_pseudo, weights):
    s = _rowdot(T, X_pseudo, weights)
    pad = N_ROWS_PAD - N_ROWS
    s2d = jnp.concatenate(
        [s.reshape(N_ROWS), jnp.zeros((pad,), jnp.float32)]
    ).reshape(N_WINDOWS, WINDOW)
    ids2d = jnp.concatenate(
        [segment_ids.astype(jnp.int32), jnp.full((pad,), DUMMY_SEG, jnp.int32)]
    ).reshape(N_WINDOWS, WINDOW)
    out = _segment_sum_sc(ids2d, s2d)
    return out.reshape(N_SEG, 1)


# speedup vs baseline: 5.3590x; 1.0183x over previous
import functools

import jax
import jax.numpy as jnp
from jax import lax
from jax.experimental import pallas as pl
from jax.experimental.pallas import tpu as pltpu
from jax.experimental.pallas import tpu_sc as plsc

N_ROWS = 320000
D_FEAT = 128
N_SEG = 10000
N_SEG_PAD = 10240

ROW_BLOCK = 20000

NUM_TILES = 16
WINDOW = 128
W_PER_TILE = 160
N_WINDOWS = NUM_TILES * W_PER_TILE
N_ROWS_PAD = N_WINDOWS * WINDOW
DUMMY_SEG = N_SEG + 8


def _matvec_body(t_ref, x_ref, w_ref, s_ref):
    v_row = jnp.dot(w_ref[...], x_ref[...], preferred_element_type=jnp.float32)
    s_ref[...] = jnp.dot(t_ref[...], v_row.T, preferred_element_type=jnp.float32)


def _rowdot(T, X_pseudo, weights):
    return pl.pallas_call(
        _matvec_body,
        grid=(N_ROWS // ROW_BLOCK,),
        in_specs=[
            pl.BlockSpec((ROW_BLOCK, D_FEAT), lambda i: (i, 0)),
            pl.BlockSpec((512, D_FEAT), lambda i: (0, 0)),
            pl.BlockSpec((1, 512), lambda i: (0, 0)),
        ],
        out_specs=pl.BlockSpec((ROW_BLOCK, 1), lambda i: (i, 0)),
        out_shape=jax.ShapeDtypeStruct((N_ROWS, 1), jnp.float32),
    )(T, X_pseudo, weights)


def _segsum_tec(ids_hbm, s_hbm, out_hbm, idx_v, upd_v, zero_v, stage_v, acc_shared):
    tid = lax.axis_index("s")

    for i in range(640 // 16):
        zero_v[pl.ds(i * 16, 16)] = jnp.zeros((16,), jnp.float32)
    pltpu.sync_copy(zero_v, acc_shared.at[pl.ds(tid * 640, 640)])
    plsc.subcore_barrier()

    pltpu.sync_copy(ids_hbm.at[pl.ds(tid * W_PER_TILE, W_PER_TILE)], idx_v)
    pltpu.sync_copy(s_hbm.at[pl.ds(tid * W_PER_TILE, W_PER_TILE)], upd_v)

    def body(w, carry):
        pltpu.sync_copy(upd_v.at[w], acc_shared.at[idx_v.at[w]], add=True)
        return carry

    lax.fori_loop(0, W_PER_TILE, body, 0)
    plsc.subcore_barrier()

    @pl.when(tid == 0)
    def _():
        pltpu.sync_copy(acc_shared.at[pl.ds(0, N_SEG)], stage_v)
        pltpu.sync_copy(stage_v, out_hbm)


def _segment_sum_sc(ids2d, s2d):
    mesh = plsc.VectorSubcoreMesh(
        core_axis_name="c", subcore_axis_name="s", num_cores=1
    )
    f = pl.kernel(
        _segsum_tec,
        out_type=jax.ShapeDtypeStruct((N_SEG,), jnp.float32),
        mesh=mesh,
        scratch_types=[
            pltpu.VMEM((W_PER_TILE, WINDOW), jnp.int32),
            pltpu.VMEM((W_PER_TILE, WINDOW), jnp.float32),
            pltpu.VMEM((640,), jnp.float32),
            pltpu.VMEM((N_SEG,), jnp.float32),
            pltpu.VMEM_SHARED((N_SEG_PAD,), jnp.float32),
        ],
    )
    return f(ids2d, s2d)


def kernel(T, segment_ids, X_pseudo, weights):
    s = _rowdot(T, X_pseudo, weights)
    pad = N_ROWS_PAD - N_ROWS
    s2d = jnp.concatenate(
        [s.reshape(N_ROWS), jnp.zeros((pad,), jnp.float32)]
    ).reshape(N_WINDOWS, WINDOW)
    ids2d = jnp.concatenate(
        [segment_ids.astype(jnp.int32), jnp.full((pad,), DUMMY_SEG, jnp.int32)]
    ).reshape(N_WINDOWS, WINDOW)
    out = _segment_sum_sc(ids2d, s2d)
    return out.reshape(N_SEG, 1)


# speedup vs baseline: 10.0886x; 1.8825x over previous
import functools

import jax
import jax.numpy as jnp
from jax import lax
from jax.experimental import pallas as pl
from jax.experimental.pallas import tpu as pltpu
from jax.experimental.pallas import tpu_sc as plsc

N_ROWS = 320000
D_FEAT = 128
N_SEG = 10000
N_SEG_PAD = 10240

W_BLOCK = 125
N_STEPS_TC = 20

NUM_TILES = 16
WINDOW = 128
N_WINDOWS_REAL = N_ROWS // WINDOW
W_PER_TILE = 160
N_WINDOWS = NUM_TILES * W_PER_TILE


def _matvec_body(t_ref, x_ref, w_ref, s_ref):
    v_row = jnp.dot(w_ref[...], x_ref[...], preferred_element_type=jnp.float32)
    s_ref[...] = jnp.sum(t_ref[...] * v_row[0][None, None, :], axis=2)[None]


def _rowdot(T3, X_pseudo, weights):
    return pl.pallas_call(
        _matvec_body,
        grid=(N_STEPS_TC,),
        in_specs=[
            pl.BlockSpec((W_BLOCK, 128, D_FEAT), lambda i: (i, 0, 0)),
            pl.BlockSpec((512, D_FEAT), lambda i: (0, 0)),
            pl.BlockSpec((1, 512), lambda i: (0, 0)),
        ],
        out_specs=pl.BlockSpec((1, W_BLOCK, WINDOW), lambda i: (i, 0, 0)),
        out_shape=jax.ShapeDtypeStruct(
            (N_STEPS_TC, W_BLOCK, WINDOW), jnp.float32
        ),
    )(T3, X_pseudo, weights)


def _segsum_tec(ids_hbm, s_hbm, out_hbm, idx_v, upd_v, zero_v, stage_v, acc_shared):
    tid = lax.axis_index("s")

    for i in range(640 // 16):
        zero_v[pl.ds(i * 16, 16)] = jnp.zeros((16,), jnp.float32)
    pltpu.sync_copy(zero_v, acc_shared.at[pl.ds(tid * 640, 640)])
    plsc.subcore_barrier()

    pltpu.sync_copy(ids_hbm.at[pl.ds(tid * W_PER_TILE, W_PER_TILE)], idx_v)
    pltpu.sync_copy(s_hbm.at[pl.ds(tid * W_PER_TILE, W_PER_TILE)], upd_v)

    n_w = jnp.minimum(W_PER_TILE, N_WINDOWS_REAL - tid * W_PER_TILE)

    def body(w, carry):
        pltpu.sync_copy(upd_v.at[w], acc_shared.at[idx_v.at[w]], add=True)
        return carry

    lax.fori_loop(0, n_w, body, 0)
    plsc.subcore_barrier()

    @pl.when(tid == 0)
    def _():
        pltpu.sync_copy(acc_shared.at[pl.ds(0, N_SEG)], stage_v)
        pltpu.sync_copy(stage_v, out_hbm)


def _segment_sum_sc(ids2d, s2d):
    mesh = plsc.VectorSubcoreMesh(
        core_axis_name="c", subcore_axis_name="s", num_cores=1
    )
    f = pl.kernel(
        _segsum_tec,
        out_type=jax.ShapeDtypeStruct((N_SEG,), jnp.float32),
        mesh=mesh,
        scratch_types=[
            pltpu.VMEM((W_PER_TILE, WINDOW), jnp.int32),
            pltpu.VMEM((W_PER_TILE, WINDOW), jnp.float32),
            pltpu.VMEM((640,), jnp.float32),
            pltpu.VMEM((N_SEG,), jnp.float32),
            pltpu.VMEM_SHARED((N_SEG_PAD,), jnp.float32),
        ],
    )
    return f(ids2d, s2d)


def kernel(T, segment_ids, X_pseudo, weights):
    T3 = T.reshape(N_WINDOWS_REAL, WINDOW, D_FEAT)
    s3d = _rowdot(T3, X_pseudo, weights)
    pad = N_WINDOWS - N_WINDOWS_REAL
    s2d = jnp.concatenate(
        [s3d.reshape(N_WINDOWS_REAL, WINDOW), jnp.zeros((pad, WINDOW), jnp.float32)]
    )
    ids2d = jnp.concatenate(
        [
            segment_ids.astype(jnp.int32).reshape(N_WINDOWS_REAL, WINDOW),
            jnp.zeros((pad, WINDOW), jnp.int32),
        ]
    )
    out = _segment_sum_sc(ids2d, s2d)
    return out.reshape(N_SEG, 1)


# speedup vs baseline: 10.9336x; 1.0838x over previous
import functools

import jax
import jax.numpy as jnp
from jax import lax
from jax.experimental import pallas as pl
from jax.experimental.pallas import tpu as pltpu
from jax.experimental.pallas import tpu_sc as plsc

N_ROWS = 320000
D_FEAT = 128
N_SEG = 10000
N_SEG_PAD = 10240

W_BLOCK = 125
N_STEPS_TC = 20

NUM_TILES = 16
WINDOW = 128
N_WINDOWS_REAL = N_ROWS // WINDOW
W_PER_TILE = 160
N_WINDOWS = NUM_TILES * W_PER_TILE


def _matvec_body(ta_ref, tb_ref, x_ref, w_ref, sa_ref, sb_ref):
    v_row = jnp.dot(w_ref[...], x_ref[...], preferred_element_type=jnp.float32)
    v = v_row[0][None, None, :]
    sa_ref[...] = jnp.sum(ta_ref[...] * v, axis=2)[None]
    sb_ref[...] = jnp.sum(tb_ref[...] * v, axis=2)[None]


def _rowdot(T3, X_pseudo, weights):
    half = N_STEPS_TC // 2
    outs = pl.pallas_call(
        _matvec_body,
        grid=(half,),
        in_specs=[
            pl.BlockSpec((W_BLOCK, 128, D_FEAT), lambda i: (i, 0, 0)),
            pl.BlockSpec((W_BLOCK, 128, D_FEAT), lambda i: (half + i, 0, 0)),
            pl.BlockSpec((512, D_FEAT), lambda i: (0, 0)),
            pl.BlockSpec((1, 512), lambda i: (0, 0)),
        ],
        out_specs=[
            pl.BlockSpec((1, W_BLOCK, WINDOW), lambda i: (i, 0, 0)),
            pl.BlockSpec((1, W_BLOCK, WINDOW), lambda i: (i, 0, 0)),
        ],
        out_shape=[
            jax.ShapeDtypeStruct((half, W_BLOCK, WINDOW), jnp.float32),
            jax.ShapeDtypeStruct((half, W_BLOCK, WINDOW), jnp.float32),
        ],
    )(T3, T3, X_pseudo, weights)
    return jnp.concatenate(outs, axis=0)


def _segsum_tec(
    ids_hbm, s_hbm, out_hbm, idx_v, upd_v, zero_v, stage_v, acc_shared,
    sem_stage, sem_scat
):
    tid = lax.axis_index("s")

    d_idx = pltpu.async_copy(
        ids_hbm.at[pl.ds(tid * W_PER_TILE, W_PER_TILE)], idx_v, sem_stage
    )
    d_upd = pltpu.async_copy(
        s_hbm.at[pl.ds(tid * W_PER_TILE, W_PER_TILE)], upd_v, sem_stage
    )

    for i in range(640 // 16):
        zero_v[pl.ds(i * 16, 16)] = jnp.zeros((16,), jnp.float32)
    pltpu.sync_copy(zero_v, acc_shared.at[pl.ds(tid * 640, 640)])
    plsc.subcore_barrier()
    d_idx.wait()
    d_upd.wait()

    n_w = jnp.minimum(W_PER_TILE, N_WINDOWS_REAL - tid * W_PER_TILE)
    G = 4

    def body(g, carry):
        descs = [
            pltpu.async_copy(
                upd_v.at[g * G + j],
                acc_shared.at[idx_v.at[g * G + j]],
                sem_scat,
                add=True,
            )
            for j in range(G)
        ]
        for d in descs:
            d.wait()
        return carry

    lax.fori_loop(0, n_w // G, body, 0)
    plsc.subcore_barrier()

    @pl.when(tid == 0)
    def _():
        pltpu.sync_copy(acc_shared.at[pl.ds(0, N_SEG)], stage_v)
        pltpu.sync_copy(stage_v, out_hbm)


def _segment_sum_sc(ids2d, s2d):
    mesh = plsc.VectorSubcoreMesh(
        core_axis_name="c", subcore_axis_name="s", num_cores=1
    )
    f = pl.kernel(
        _segsum_tec,
        out_type=jax.ShapeDtypeStruct((N_SEG,), jnp.float32),
        mesh=mesh,
        scratch_types=[
            pltpu.VMEM((W_PER_TILE, WINDOW), jnp.int32),
            pltpu.VMEM((W_PER_TILE, WINDOW), jnp.float32),
            pltpu.VMEM((640,), jnp.float32),
            pltpu.VMEM((N_SEG,), jnp.float32),
            pltpu.VMEM_SHARED((N_SEG_PAD,), jnp.float32),
            pltpu.SemaphoreType.DMA,
            pltpu.SemaphoreType.DMA,
        ],
    )
    return f(ids2d, s2d)


def kernel(T, segment_ids, X_pseudo, weights):
    T3 = T.reshape(N_WINDOWS_REAL, WINDOW, D_FEAT)
    s3d = _rowdot(T3, X_pseudo, weights)
    pad = N_WINDOWS - N_WINDOWS_REAL
    s2d = jnp.concatenate(
        [s3d.reshape(N_WINDOWS_REAL, WINDOW), jnp.zeros((pad, WINDOW), jnp.float32)]
    )
    ids2d = jnp.concatenate(
        [
            segment_ids.astype(jnp.int32).reshape(N_WINDOWS_REAL, WINDOW),
            jnp.zeros((pad, WINDOW), jnp.int32),
        ]
    )
    out = _segment_sum_sc(ids2d, s2d)
    return out.reshape(N_SEG, 1)


# speedup vs baseline: 11.0710x; 1.0126x over previous
import functools

import jax
import jax.numpy as jnp
from jax import lax
from jax.experimental import pallas as pl
from jax.experimental.pallas import tpu as pltpu
from jax.experimental.pallas import tpu_sc as plsc

N_ROWS = 320000
D_FEAT = 128
N_SEG = 10000
N_SEG_PAD = 10240

W_BLOCK = 125
N_STEPS_TC = 20

NUM_TILES = 16
WINDOW = 128
N_WINDOWS_REAL = N_ROWS // WINDOW
W_PER_TILE = 160
N_WINDOWS = NUM_TILES * W_PER_TILE


def _matvec_body(ta_ref, tb_ref, x_ref, w_ref, sa_ref, sb_ref):
    v_row = jnp.dot(w_ref[...], x_ref[...], preferred_element_type=jnp.float32)
    v = v_row[0][None, None, :]
    sa_ref[...] = jnp.sum(ta_ref[...] * v, axis=2)[None]
    sb_ref[...] = jnp.sum(tb_ref[...] * v, axis=2)[None]


def _rowdot(T3, X_pseudo, weights):
    half = N_STEPS_TC // 2
    outs = pl.pallas_call(
        _matvec_body,
        grid=(half,),
        in_specs=[
            pl.BlockSpec((W_BLOCK, 128, D_FEAT), lambda i: (i, 0, 0)),
            pl.BlockSpec((W_BLOCK, 128, D_FEAT), lambda i: (half + i, 0, 0)),
            pl.BlockSpec((512, D_FEAT), lambda i: (0, 0)),
            pl.BlockSpec((1, 512), lambda i: (0, 0)),
        ],
        out_specs=[
            pl.BlockSpec((1, W_BLOCK, WINDOW), lambda i: (i, 0, 0)),
            pl.BlockSpec((1, W_BLOCK, WINDOW), lambda i: (i, 0, 0)),
        ],
        out_shape=[
            jax.ShapeDtypeStruct((half, W_BLOCK, WINDOW), jnp.float32),
            jax.ShapeDtypeStruct((half, W_BLOCK, WINDOW), jnp.float32),
        ],
    )(T3, T3, X_pseudo, weights)
    return jnp.concatenate(outs, axis=0)


def _segsum_tec(
    ids_hbm, s_hbm, out_hbm, idx_v, upd_v, zero_v, stage_v, acc_shared,
    sem_stage, sem_scat
):
    tid = lax.axis_index("s")

    d_idx = pltpu.async_copy(
        ids_hbm.at[pl.ds(tid * W_PER_TILE, W_PER_TILE)], idx_v, sem_stage
    )
    d_upd = pltpu.async_copy(
        s_hbm.at[pl.ds(tid * W_PER_TILE, W_PER_TILE)], upd_v, sem_stage
    )

    for i in range(640 // 16):
        zero_v[pl.ds(i * 16, 16)] = jnp.zeros((16,), jnp.float32)
    pltpu.sync_copy(zero_v, acc_shared.at[pl.ds(tid * 640, 640)])
    plsc.subcore_barrier()
    d_idx.wait()
    d_upd.wait()

    n_w = jnp.minimum(W_PER_TILE, N_WINDOWS_REAL - tid * W_PER_TILE)
    G = 20

    def body(g, carry):
        descs = [
            pltpu.async_copy(
                upd_v.at[g * G + j],
                acc_shared.at[idx_v.at[g * G + j]],
                sem_scat,
                add=True,
            )
            for j in range(G)
        ]
        for d in descs:
            d.wait()
        return carry

    lax.fori_loop(0, n_w // G, body, 0)
    plsc.subcore_barrier()

    @pl.when(tid < 15)
    def _():
        pltpu.sync_copy(acc_shared.at[pl.ds(tid * 640, 640)], stage_v)
        pltpu.sync_copy(stage_v, out_hbm.at[pl.ds(tid * 640, 640)])

    @pl.when(tid == 15)
    def _():
        pltpu.sync_copy(acc_shared.at[pl.ds(9600, 400)], stage_v.at[pl.ds(0, 400)])
        pltpu.sync_copy(stage_v.at[pl.ds(0, 400)], out_hbm.at[pl.ds(9600, 400)])


def _segment_sum_sc(ids2d, s2d):
    mesh = plsc.VectorSubcoreMesh(
        core_axis_name="c", subcore_axis_name="s", num_cores=1
    )
    f = pl.kernel(
        _segsum_tec,
        out_type=jax.ShapeDtypeStruct((N_SEG,), jnp.float32),
        mesh=mesh,
        scratch_types=[
            pltpu.VMEM((W_PER_TILE, WINDOW), jnp.int32),
            pltpu.VMEM((W_PER_TILE, WINDOW), jnp.float32),
            pltpu.VMEM((640,), jnp.float32),
            pltpu.VMEM((640,), jnp.float32),
            pltpu.VMEM_SHARED((N_SEG_PAD,), jnp.float32),
            pltpu.SemaphoreType.DMA,
            pltpu.SemaphoreType.DMA,
        ],
    )
    return f(ids2d, s2d)


def kernel(T, segment_ids, X_pseudo, weights):
    T3 = T.reshape(N_WINDOWS_REAL, WINDOW, D_FEAT)
    s3d = _rowdot(T3, X_pseudo, weights)
    pad = N_WINDOWS - N_WINDOWS_REAL
    s2d = jnp.concatenate(
        [s3d.reshape(N_WINDOWS_REAL, WINDOW), jnp.zeros((pad, WINDOW), jnp.float32)]
    )
    ids2d = jnp.concatenate(
        [
            segment_ids.astype(jnp.int32).reshape(N_WINDOWS_REAL, WINDOW),
            jnp.zeros((pad, WINDOW), jnp.int32),
        ]
    )
    out = _segment_sum_sc(ids2d, s2d)
    return out.reshape(N_SEG, 1)
